# baseline (speedup 1.0000x reference)
import functools

import jax
import jax.numpy as jnp
from jax import lax
from jax.experimental import pallas as pl
from jax.experimental.pallas import tpu as pltpu
from jax.experimental.pallas import tpu_sc as plsc

N = 10000
E = 320000
D = 128
L = 16
NC = 2
NS = 16
NP = NS * 640
CPT = D // NS
PPT = CPT // 2
EPC = E // NC
CE = 640
NCH = EPC // CE
GRP = CE // L
DSUB = CE // 80
U = 8


def _sc_body(xp_hbm, edata_hbm, dstd_hbm, valsf_hbm, p_hbm, deg_hbm,
             xpair_v, acc_v, ebuf0, ebuf1, ddst_v, vf_v, zdeg_v,
             dacc_sh, esem0, esem1, dsem):
    cid = lax.axis_index("c")
    sid = lax.axis_index("s")

    pltpu.sync_copy(xp_hbm.at[sid], xpair_v)

    zeros16 = jnp.zeros((L,), jnp.float32)

    def zacc(i, carry):
        for r in range(CPT):
            acc_v[r, pl.ds(i * L, L)] = zeros16
        return carry

    lax.fori_loop(0, NP // L, zacc, 0)

    for j in range(640 // L):
        zdeg_v[pl.ds(j * L, L)] = zeros16
    doff = pl.multiple_of(sid * 640, 8)
    pltpu.sync_copy(zdeg_v, dacc_sh.at[pl.ds(doff, 640)])
    plsc.subcore_barrier()

    idxc = [jnp.full((L,), c, jnp.int32) for c in range(CPT)]
    sh16 = jnp.full((L,), 16, jnp.int32)
    msk16 = jnp.full((L,), -65536, jnp.int32)

    def process(k, ebuf):
        def group_body(gi, carry):
            for u in range(U):
                base = (gi * U + u) * L
                src16 = ebuf[0, pl.ds(base, L)]
                dst16 = ebuf[1, pl.ds(base, L)]
                val16 = plsc.bitcast(ebuf[2, pl.ds(base, L)], jnp.float32)
                for pc in range(PPT):
                    pair = plsc.load_gather(xpair_v, [idxc[pc], src16])
                    lo = plsc.bitcast(lax.shift_left(pair, sh16),
                                      jnp.float32)
                    hi = plsc.bitcast(lax.bitwise_and(pair, msk16),
                                      jnp.float32)
                    plsc.addupdate_scatter(acc_v, [idxc[2 * pc], dst16],
                                           lo * val16)
                    plsc.addupdate_scatter(acc_v, [idxc[2 * pc + 1], dst16],
                                           hi * val16)
            return carry

        lax.fori_loop(0, GRP // U, group_body, 0)

        @pl.when(sid == lax.rem(k, NS))
        def _():
            @pl.when(k >= NS)
            def _():
                for j in range(DSUB):
                    pltpu.make_async_copy(
                        vf_v.at[j], dacc_sh.at[ddst_v.at[j]], dsem).wait()

            pltpu.sync_copy(dstd_hbm.at[cid, k], ddst_v)
            pltpu.sync_copy(valsf_hbm.at[cid, k], vf_v)
            for j in range(DSUB):
                pltpu.async_copy(vf_v.at[j], dacc_sh.at[ddst_v.at[j]],
                                 dsem, add=True)

    pltpu.async_copy(edata_hbm.at[cid, 0], ebuf0, esem0)

    def chunk_iter(k, carry):
        @pl.when(k % 2 == 0)
        def _():
            @pl.when(k < NCH - 1)
            def _():
                pltpu.async_copy(edata_hbm.at[cid, k + 1], ebuf1, esem1)
            pltpu.make_async_copy(edata_hbm.at[cid, k], ebuf0, esem0).wait()
            process(k, ebuf0)

        @pl.when(k % 2 == 1)
        def _():
            @pl.when(k < NCH - 1)
            def _():
                pltpu.async_copy(edata_hbm.at[cid, k + 1], ebuf0, esem0)
            pltpu.make_async_copy(edata_hbm.at[cid, k], ebuf1, esem1).wait()
            process(k, ebuf1)

        return carry

    lax.fori_loop(0, NCH, chunk_iter, 0)

    for j in range(DSUB):
        pltpu.make_async_copy(
            vf_v.at[j], dacc_sh.at[ddst_v.at[j]], dsem).wait()
    plsc.subcore_barrier()

    coff = pl.multiple_of(sid * CPT, 8)
    pltpu.sync_copy(acc_v, p_hbm.at[cid, pl.ds(coff, CPT)])
    pltpu.sync_copy(dacc_sh.at[pl.ds(doff, 640)],
                    deg_hbm.at[cid, pl.ds(doff, 640)])


_sc_scatter = functools.partial(
    pl.kernel,
    out_type=[
        jax.ShapeDtypeStruct((NC, D, NP), jnp.float32),
        jax.ShapeDtypeStruct((NC, NP), jnp.float32),
    ],
    mesh=plsc.VectorSubcoreMesh(core_axis_name="c", subcore_axis_name="s"),
    compiler_params=pltpu.CompilerParams(needs_layout_passes=False),
    scratch_types=[
        pltpu.VMEM((PPT, N), jnp.int32),
        pltpu.VMEM((CPT, NP), jnp.float32),
        pltpu.VMEM((3, CE), jnp.int32),
        pltpu.VMEM((3, CE), jnp.int32),
        pltpu.VMEM((DSUB, 80), jnp.int32),
        pltpu.VMEM((DSUB, 80), jnp.float32),
        pltpu.VMEM((640,), jnp.float32),
        pltpu.VMEM_SHARED((NP,), jnp.float32),
        pltpu.SemaphoreType.DMA,
        pltpu.SemaphoreType.DMA,
        pltpu.SemaphoreType.DMA,
    ],
)(_sc_body)


def _mm_body(p0_ref, p1_ref, d0_ref, d1_ref, w_ref, b_ref, o_ref):
    ht = p0_ref[...] + p1_ref[...]
    dd = d0_ref[...] + d1_ref[...]
    o_ref[...] = (lax.dot_general(ht, w_ref[...], (((0,), (1,)), ((), ())),
                                  preferred_element_type=jnp.float32)
                  + dd * b_ref[...])


_R = 2048


def _tc_matmul(p0, p1, d0, d1, w, b2):
    return pl.pallas_call(
        _mm_body,
        grid=(NP // _R,),
        in_specs=[
            pl.BlockSpec((D, _R), lambda i: (0, i)),
            pl.BlockSpec((D, _R), lambda i: (0, i)),
            pl.BlockSpec((_R, 1), lambda i: (i, 0)),
            pl.BlockSpec((_R, 1), lambda i: (i, 0)),
            pl.BlockSpec((D, D), lambda i: (0, 0)),
            pl.BlockSpec((1, D), lambda i: (0, 0)),
        ],
        out_specs=pl.BlockSpec((_R, D), lambda i: (i, 0)),
        out_shape=jax.ShapeDtypeStruct((NP, D), jnp.float32),
    )(p0, p1, d0, d1, w, b2)


def kernel(x, edge_index, adj_vals, W, b):
    ei = edge_index.astype(jnp.int32)
    xb = jax.lax.bitcast_convert_type(
        x.astype(jnp.bfloat16).reshape(N, D // 2, 2), jnp.int32)
    xp = jnp.transpose(xb, (1, 0)).reshape(NS, PPT, N)
    vals_i = jax.lax.bitcast_convert_type(adj_vals, jnp.int32)
    edata = jnp.stack([
        ei[1].reshape(NC, NCH, CE),
        ei[0].reshape(NC, NCH, CE),
        vals_i.reshape(NC, NCH, CE),
    ], axis=2)
    dstd = ei[0].reshape(NC, NCH, DSUB, 80)
    valsf = adj_vals.reshape(NC, NCH, DSUB, 80)
    P, deg = _sc_scatter(xp, edata, dstd, valsf)
    out = _tc_matmul(P[0], P[1], deg[0][:, None], deg[1][:, None],
                     W, b[None, :])
    return out[:N]


# speedup vs baseline: 2.6458x; 2.6458x over previous
import functools

import jax
import jax.numpy as jnp
from jax import lax
from jax.experimental import pallas as pl
from jax.experimental.pallas import tpu as pltpu
from jax.experimental.pallas import tpu_sc as plsc

N = 10000
E = 320000
D = 128
L = 16
NC = 2
NS = 16
NW = NC * NS
NP = NS * 640
RPT = NP // NS
EPW = E // NW
C = 80
NCHUNK = EPW // C
KSUP = 25
NSUP = NCHUNK // KSUP
GROUPS = C // L


def _scale_rows(rows_v, vals_v, i):
    def group_body(g, carry2):
        vv = vals_v[i, pl.ds(g * L, L)]
        for r in range(L):
            s = vv.at[jnp.full((L,), r, jnp.int32)].get(
                mode="promise_in_bounds")
            row = g * L + r
            for j in range(D // L):
                sl = pl.ds(j * L, L)
                rows_v[row, sl] = rows_v[row, sl] * s
        return carry2

    lax.fori_loop(0, GROUPS, group_body, 0)


def _sc_body(x_hbm, src_hbm, dst_hbm, vals_hbm, p_hbm, deg_hbm,
             src_v, dst_v, vals_v, rows0_v, rows1_v, rows2_v, zdeg_v,
             acc_sh, dacc_sh, gsem0, gsem1, gsem2, ssem0, ssem1, ssem2, dsem):
    cid = lax.axis_index("c")
    sid = lax.axis_index("s")
    wid = sid * NC + cid

    zeros16 = jnp.zeros((L,), jnp.float32)

    def zrow(r, carry):
        for j in range(D // L):
            rows0_v[r, pl.ds(j * L, L)] = zeros16
        return carry

    lax.fori_loop(0, C, zrow, 0)
    for j in range(RPT // L):
        zdeg_v[pl.ds(j * L, L)] = zeros16

    row0 = pl.multiple_of(sid * RPT, 8)
    for k in range(RPT // C):
        pltpu.sync_copy(rows0_v, acc_sh.at[pl.ds(row0 + k * C, C)])
    pltpu.sync_copy(zdeg_v, dacc_sh.at[pl.ds(row0, RPT)])
    plsc.subcore_barrier()

    def step(i, cur_rows, cur_gsem, cur_ssem, nxt_rows, nxt_gsem, nxt_ssem):
        @pl.when(i >= 2)
        def _():
            pltpu.make_async_copy(
                nxt_rows, acc_sh.at[pl.ds(0, C)], nxt_ssem).wait()

        @pl.when(i < KSUP - 1)
        def _():
            pltpu.async_copy(x_hbm.at[src_v.at[i + 1]], nxt_rows, nxt_gsem)

        pltpu.make_async_copy(
            x_hbm.at[src_v.at[i]], cur_rows, cur_gsem).wait()

        _scale_rows(cur_rows, vals_v, i)

        pltpu.async_copy(cur_rows, acc_sh.at[dst_v.at[i]], cur_ssem, add=True)
        pltpu.async_copy(vals_v.at[i], dacc_sh.at[dst_v.at[i]], dsem, add=True)

    def super_body(sbi, carry):
        pltpu.sync_copy(src_hbm.at[wid, sbi], src_v)
        pltpu.sync_copy(dst_hbm.at[wid, sbi], dst_v)
        pltpu.sync_copy(vals_hbm.at[wid, sbi], vals_v)

        pltpu.async_copy(x_hbm.at[src_v.at[0]], rows0_v, gsem0)

        rings = [(rows0_v, gsem0, ssem0), (rows1_v, gsem1, ssem1),
                 (rows2_v, gsem2, ssem2)]

        def chunk_iter(i, carry1):
            for p in range(3):
                @pl.when(i % 3 == p)
                def _(p=p):
                    cur = rings[p]
                    nxt = rings[(p + 1) % 3]
                    step(i, *cur, *nxt)

            return carry1

        lax.fori_loop(0, KSUP, chunk_iter, 0)

        for i in (KSUP - 2, KSUP - 1):
            rv, _, sv = rings[i % 3]
            pltpu.make_async_copy(rv, acc_sh.at[pl.ds(0, C)], sv).wait()

        def drain_deg(i, carry2):
            pltpu.make_async_copy(
                vals_v.at[0], dacc_sh.at[dst_v.at[0]], dsem).wait()
            return carry2

        lax.fori_loop(0, KSUP, drain_deg, 0)
        return carry

    lax.fori_loop(0, NSUP, super_body, 0)
    plsc.subcore_barrier()

    pltpu.sync_copy(acc_sh.at[pl.ds(row0, RPT)], p_hbm.at[cid, pl.ds(row0, RPT)])
    pltpu.sync_copy(dacc_sh.at[pl.ds(row0, RPT)], deg_hbm.at[cid, pl.ds(row0, RPT)])


_sc_scatter = functools.partial(
    pl.kernel,
    out_type=[
        jax.ShapeDtypeStruct((NC, NP, D), jnp.float32),
        jax.ShapeDtypeStruct((NC, NP), jnp.float32),
    ],
    mesh=plsc.VectorSubcoreMesh(core_axis_name="c", subcore_axis_name="s"),
    scratch_types=[
        pltpu.VMEM((KSUP, C), jnp.int32),
        pltpu.VMEM((KSUP, C), jnp.int32),
        pltpu.VMEM((KSUP, C), jnp.float32),
        pltpu.VMEM((C, D), jnp.float32),
        pltpu.VMEM((C, D), jnp.float32),
        pltpu.VMEM((C, D), jnp.float32),
        pltpu.VMEM((RPT,), jnp.float32),
        pltpu.VMEM_SHARED((NP, D), jnp.float32),
        pltpu.VMEM_SHARED((NP,), jnp.float32),
        pltpu.SemaphoreType.DMA,
        pltpu.SemaphoreType.DMA,
        pltpu.SemaphoreType.DMA,
        pltpu.SemaphoreType.DMA,
        pltpu.SemaphoreType.DMA,
        pltpu.SemaphoreType.DMA,
        pltpu.SemaphoreType.DMA,
    ],
)(_sc_body)


def _mm_body(p0_ref, p1_ref, d0_ref, d1_ref, wt_ref, b_ref, o_ref):
    h = p0_ref[...] + p1_ref[...]
    dd = d0_ref[...] + d1_ref[...]
    o_ref[...] = (jnp.dot(h, wt_ref[...], preferred_element_type=jnp.float32)
                  + dd * b_ref[...])


_R = 2048


def _tc_matmul(p0, p1, d0, d1, wt, b2):
    return pl.pallas_call(
        _mm_body,
        grid=(NP // _R,),
        in_specs=[
            pl.BlockSpec((_R, D), lambda i: (i, 0)),
            pl.BlockSpec((_R, D), lambda i: (i, 0)),
            pl.BlockSpec((_R, 1), lambda i: (i, 0)),
            pl.BlockSpec((_R, 1), lambda i: (i, 0)),
            pl.BlockSpec((D, D), lambda i: (0, 0)),
            pl.BlockSpec((1, D), lambda i: (0, 0)),
        ],
        out_specs=pl.BlockSpec((_R, D), lambda i: (i, 0)),
        out_shape=jax.ShapeDtypeStruct((NP, D), jnp.float32),
    )(p0, p1, d0, d1, wt, b2)


def kernel(x, edge_index, adj_vals, W, b):
    src = edge_index[1].astype(jnp.int32).reshape(NW, NSUP, KSUP, C)
    dst = edge_index[0].astype(jnp.int32).reshape(NW, NSUP, KSUP, C)
    vals = adj_vals.reshape(NW, NSUP, KSUP, C)
    P, deg = _sc_scatter(x, src, dst, vals)
    out = _tc_matmul(P[0], P[1], deg[0][:, None], deg[1][:, None],
                     W.T, b[None, :])
    return out[:N]
